# Initial kernel scaffold; baseline (speedup 1.0000x reference)
import jax
import jax.numpy as jnp
from jax.experimental import pallas as pl


def kernel(vals, distances):
    raise NotImplementedError("write your pallas kernel here")


# speedup vs baseline: 4.2058x; 4.2058x over previous
import functools

import jax
import jax.numpy as jnp
from jax import lax
from jax.experimental import pallas as pl
from jax.experimental.pallas import tpu as pltpu
from jax.experimental.pallas import tpu_sc as plsc

B = 32
S = 8
MAX_K = 32
V = 100000
TEMPERATURE = 10.0

R = B * S
NC = 2
NS = 16
NW = NC * NS
ROWS_PER_W = R // NW
L = 16


def _body(vals_hbm, dist_hbm, out_hbm, vals_v, dist_v, row_buf):
    wid = lax.axis_index("s") * NC + lax.axis_index("c")
    base = wid * ROWS_PER_W

    pltpu.sync_copy(vals_hbm.at[pl.ds(base * MAX_K, ROWS_PER_W * MAX_K)], vals_v)
    pltpu.sync_copy(dist_hbm.at[pl.ds(base * MAX_K, ROWS_PER_W * MAX_K)], dist_v)

    zeros16 = jnp.zeros((L,), jnp.float32)

    def zero_step(i, carry):
        for j in range(10):
            row_buf[pl.ds(i * (10 * L) + j * L, L)] = zeros16
        return carry

    lax.fori_loop(0, V // (10 * L), zero_step, 0)

    lane_iota = lax.iota(jnp.int32, L)

    for r in range(ROWS_PER_W):
        idx0 = vals_v[pl.ds(r * MAX_K, L)]
        idx1 = vals_v[pl.ds(r * MAX_K + L, L)]
        d0 = dist_v[pl.ds(r * MAX_K, L)]
        d1 = dist_v[pl.ds(r * MAX_K + L, L)]

        e0 = jnp.exp(d0 * (-1.0 / TEMPERATURE))
        e1 = jnp.exp(d1 * (-1.0 / TEMPERATURE))
        t = e0 + e1
        for sh in (8, 4, 2, 1):
            t = t + t.at[lane_iota ^ sh].get(mode="promise_in_bounds")
        inv = 1.0 / t
        w0 = e0 * inv
        w1 = e1 * inv

        for k in range(L):
            m = lane_iota == k
            plsc.addupdate_scatter(row_buf, [idx0], w0, mask=m)
        for k in range(L):
            m = lane_iota == k
            plsc.addupdate_scatter(row_buf, [idx1], w1, mask=m)

        pltpu.sync_copy(row_buf, out_hbm.at[base + r])
        plsc.store_scatter(row_buf, [idx0], zeros16)
        plsc.store_scatter(row_buf, [idx1], zeros16)


@functools.partial(
    pl.kernel,
    mesh=plsc.VectorSubcoreMesh(core_axis_name="c", subcore_axis_name="s"),
    out_type=jax.ShapeDtypeStruct((R, V), jnp.float32),
    scratch_types=[
        pltpu.VMEM((ROWS_PER_W * MAX_K,), jnp.int32),
        pltpu.VMEM((ROWS_PER_W * MAX_K,), jnp.float32),
        pltpu.VMEM((V,), jnp.float32),
    ],
    compiler_params=pltpu.CompilerParams(needs_layout_passes=False),
)
def _combine(vals_hbm, dist_hbm, out_hbm, vals_v, dist_v, row_buf):
    _body(vals_hbm, dist_hbm, out_hbm, vals_v, dist_v, row_buf)


def kernel(vals, distances):
    vals_flat = vals.reshape(R * MAX_K).astype(jnp.int32)
    dist_flat = distances.reshape(R * MAX_K).astype(jnp.float32)
    out = _combine(vals_flat, dist_flat)
    return out.reshape(B, S, V)
